# baseline (speedup 1.0000x reference)
import functools
import math

import jax
import jax.numpy as jnp
from jax.experimental import pallas as pl
from jax.experimental.pallas import tpu as pltpu

N_NODES = 10000
N_EDGES = 160000
RATIO = 0.9


def _mm_body(x_ref, w_ref, o_ref):
    o_ref[...] = jnp.dot(x_ref[...], w_ref[...], preferred_element_type=jnp.float32)


def _pad_rows(x, bm):
    m = x.shape[0]
    mp = ((m + bm - 1) // bm) * bm
    if mp != m:
        x = jnp.pad(x, ((0, mp - m), (0, 0)))
    return x, mp


def mm(x, W, bm=1024):
    m, k = x.shape
    n = W.shape[1]
    xp, mp = _pad_rows(x, bm)
    out = pl.pallas_call(
        _mm_body,
        grid=(mp // bm,),
        in_specs=[
            pl.BlockSpec((bm, k), lambda i: (i, 0)),
            pl.BlockSpec((k, n), lambda i: (0, 0)),
        ],
        out_specs=pl.BlockSpec((bm, n), lambda i: (i, 0)),
        out_shape=jax.ShapeDtypeStruct((mp, n), jnp.float32),
    )(xp, W)
    return out[:m]


def _head_body(h_ref, w_ref, b_ref, o_ref, acc_ref, *, n_rows, n_cls):
    i = pl.program_id(0)
    nsteps = pl.num_programs(0)

    @pl.when(i == 0)
    def _():
        acc_ref[...] = jnp.zeros_like(acc_ref)

    acc_ref[...] += jnp.sum(h_ref[...], axis=0, keepdims=True)

    @pl.when(i == nsteps - 1)
    def _():
        g = acc_ref[...] / n_rows
        logits = jnp.dot(g, w_ref[...], preferred_element_type=jnp.float32)
        logits = logits + b_ref[...]
        col = jax.lax.broadcasted_iota(jnp.int32, logits.shape, 1)
        valid = col < n_cls
        neg = jnp.float32(-1e30)
        lm = jnp.where(valid, logits, neg)
        mx = jnp.max(lm)
        ex = jnp.where(valid, jnp.exp(lm - mx), 0.0)
        lse = jnp.log(jnp.sum(ex))
        o_ref[...] = lm - mx - lse


def head(h, Wlin, blin, bm=1024):
    m, k = h.shape
    n_cls = Wlin.shape[1]
    hp, mp = _pad_rows(h, bm)
    wp = jnp.zeros((k, 128), jnp.float32).at[:, :n_cls].set(Wlin)
    bp = jnp.zeros((1, 128), jnp.float32).at[0, :n_cls].set(blin)
    out = pl.pallas_call(
        functools.partial(_head_body, n_rows=m, n_cls=n_cls),
        grid=(mp // bm,),
        in_specs=[
            pl.BlockSpec((bm, k), lambda i: (i, 0)),
            pl.BlockSpec((k, 128), lambda i: (0, 0)),
            pl.BlockSpec((1, 128), lambda i: (0, 0)),
        ],
        out_specs=pl.BlockSpec((1, 128), lambda i: (0, 0)),
        out_shape=jax.ShapeDtypeStruct((1, 128), jnp.float32),
        scratch_shapes=[pltpu.VMEM((1, k), jnp.float32)],
    )(hp, wp, bp)
    return out[:, :n_cls]


def _gcn_conv(x, src, dst, w_edge, W, b, n):
    h = mm(x, W)
    loop = jnp.arange(n, dtype=src.dtype)
    src_c = jnp.concatenate([src, loop])
    dst_c = jnp.concatenate([dst, loop])
    w_c = jnp.concatenate([w_edge, jnp.ones((n,), x.dtype)])
    deg = jax.ops.segment_sum(w_c, dst_c, num_segments=n)
    dinv = jnp.where(deg > 0, 1.0 / jnp.sqrt(deg), 0.0)
    norm = dinv[src_c] * dinv[dst_c] * w_c
    out = jax.ops.segment_sum(h[src_c] * norm[:, None], dst_c, num_segments=n)
    return out + b


def _topk_pool(x, src, dst, w_edge, p, n):
    score = jnp.tanh((x @ p) / jnp.linalg.norm(p))
    k = int(math.ceil(RATIO * n))
    vals, perm = jax.lax.top_k(score, k)
    x_new = x[perm] * vals[:, None]
    mapping = (
        jnp.full((n,), -1, dtype=jnp.int32)
        .at[perm]
        .set(jnp.arange(k, dtype=jnp.int32))
    )
    ms = mapping[src]
    md = mapping[dst]
    valid = (ms >= 0) & (md >= 0) & (w_edge > 0)
    new_src = jnp.where(valid, ms, 0)
    new_dst = jnp.where(valid, md, 0)
    new_w = valid.astype(x.dtype)
    return x_new, new_src, new_dst, new_w, perm, k


def kernel(x, edge_index, W1, b1, W2, b2, W3, b3, p1, W4, b4, W5, b5, W6, b6, p2, Wlin, blin):
    src = edge_index[0]
    dst = edge_index[1]
    w_e = jnp.ones((N_EDGES,), jnp.float32)

    h = jax.nn.relu(_gcn_conv(x, src, dst, w_e, W1, b1, N_NODES))
    h = jax.nn.relu(_gcn_conv(h, src, dst, w_e, W2, b2, N_NODES))
    h = jax.nn.relu(_gcn_conv(h, src, dst, w_e, W3, b3, N_NODES))
    h, src, dst, w_e, perm1, n1 = _topk_pool(h, src, dst, w_e, p1, N_NODES)
    h = jax.nn.relu(_gcn_conv(h, src, dst, w_e, W4, b4, n1))
    h = jax.nn.relu(_gcn_conv(h, src, dst, w_e, W5, b5, n1))
    h = jax.nn.relu(_gcn_conv(h, src, dst, w_e, W6, b6, n1))
    h, src, dst, w_e, perm2, n2 = _topk_pool(h, src, dst, w_e, p2, n1)

    logits = head(h, Wlin, blin)
    return (logits, perm1, perm2)


# speedup vs baseline: 1.0056x; 1.0056x over previous
import functools
import math

import jax
import jax.numpy as jnp
from jax.experimental import pallas as pl
from jax.experimental.pallas import tpu as pltpu

N_NODES = 10000
N_EDGES = 160000
RATIO = 0.9


def _mm_body(x_ref, w_ref, o_ref):
    o_ref[...] = jnp.dot(x_ref[...], w_ref[...], preferred_element_type=jnp.float32)


def _pad_rows(x, bm):
    m = x.shape[0]
    mp = ((m + bm - 1) // bm) * bm
    if mp != m:
        x = jnp.pad(x, ((0, mp - m), (0, 0)))
    return x, mp


def _pick_bm(m):
    for bm in (2000, 1800, 1000, 904, 512):
        if m % bm == 0:
            return bm
    return None


def mm(x, W, bm=None):
    m, k = x.shape
    n = W.shape[1]
    if bm is None:
        bm = _pick_bm(m) or 1024
    xp, mp = _pad_rows(x, bm)
    out = pl.pallas_call(
        _mm_body,
        grid=(mp // bm,),
        in_specs=[
            pl.BlockSpec((bm, k), lambda i: (i, 0)),
            pl.BlockSpec((k, n), lambda i: (0, 0)),
        ],
        out_specs=pl.BlockSpec((bm, n), lambda i: (i, 0)),
        out_shape=jax.ShapeDtypeStruct((mp, n), jnp.float32),
    )(xp, W)
    return out[:m]


def _head_body(h_ref, w_ref, b_ref, o_ref, acc_ref, *, n_rows, n_cls):
    i = pl.program_id(0)
    nsteps = pl.num_programs(0)

    @pl.when(i == 0)
    def _():
        acc_ref[...] = jnp.zeros_like(acc_ref)

    acc_ref[...] += jnp.sum(h_ref[...], axis=0, keepdims=True)

    @pl.when(i == nsteps - 1)
    def _():
        g = acc_ref[...] / n_rows
        logits = jnp.dot(g, w_ref[...], preferred_element_type=jnp.float32)
        logits = logits + b_ref[...]
        col = jax.lax.broadcasted_iota(jnp.int32, logits.shape, 1)
        valid = col < n_cls
        neg = jnp.float32(-1e30)
        lm = jnp.where(valid, logits, neg)
        mx = jnp.max(lm)
        ex = jnp.where(valid, jnp.exp(lm - mx), 0.0)
        lse = jnp.log(jnp.sum(ex))
        o_ref[...] = lm - mx - lse


def head(h, Wlin, blin, bm=1024):
    m, k = h.shape
    n_cls = Wlin.shape[1]
    hp, mp = _pad_rows(h, bm)
    wp = jnp.zeros((k, 128), jnp.float32).at[:, :n_cls].set(Wlin)
    bp = jnp.zeros((1, 128), jnp.float32).at[0, :n_cls].set(blin)
    out = pl.pallas_call(
        functools.partial(_head_body, n_rows=m, n_cls=n_cls),
        grid=(mp // bm,),
        in_specs=[
            pl.BlockSpec((bm, k), lambda i: (i, 0)),
            pl.BlockSpec((k, 128), lambda i: (0, 0)),
            pl.BlockSpec((1, 128), lambda i: (0, 0)),
        ],
        out_specs=pl.BlockSpec((1, 128), lambda i: (0, 0)),
        out_shape=jax.ShapeDtypeStruct((1, 128), jnp.float32),
        scratch_shapes=[pltpu.VMEM((1, k), jnp.float32)],
    )(hp, wp, bp)
    return out[:, :n_cls]


def _gcn_conv(x, src, dst, w_edge, W, b, n):
    h = mm(x, W)
    loop = jnp.arange(n, dtype=src.dtype)
    src_c = jnp.concatenate([src, loop])
    dst_c = jnp.concatenate([dst, loop])
    w_c = jnp.concatenate([w_edge, jnp.ones((n,), x.dtype)])
    deg = jax.ops.segment_sum(w_c, dst_c, num_segments=n)
    dinv = jnp.where(deg > 0, 1.0 / jnp.sqrt(deg), 0.0)
    norm = dinv[src_c] * dinv[dst_c] * w_c
    out = jax.ops.segment_sum(h[src_c] * norm[:, None], dst_c, num_segments=n)
    return out + b


def _topk_pool(x, src, dst, w_edge, p, n):
    score = jnp.tanh((x @ p) / jnp.linalg.norm(p))
    k = int(math.ceil(RATIO * n))
    vals, perm = jax.lax.top_k(score, k)
    x_new = x[perm] * vals[:, None]
    mapping = (
        jnp.full((n,), -1, dtype=jnp.int32)
        .at[perm]
        .set(jnp.arange(k, dtype=jnp.int32))
    )
    ms = mapping[src]
    md = mapping[dst]
    valid = (ms >= 0) & (md >= 0) & (w_edge > 0)
    new_src = jnp.where(valid, ms, 0)
    new_dst = jnp.where(valid, md, 0)
    new_w = valid.astype(x.dtype)
    return x_new, new_src, new_dst, new_w, perm, k


def kernel(x, edge_index, W1, b1, W2, b2, W3, b3, p1, W4, b4, W5, b5, W6, b6, p2, Wlin, blin):
    src = edge_index[0]
    dst = edge_index[1]
    w_e = jnp.ones((N_EDGES,), jnp.float32)

    h = jax.nn.relu(_gcn_conv(x, src, dst, w_e, W1, b1, N_NODES))
    h = jax.nn.relu(_gcn_conv(h, src, dst, w_e, W2, b2, N_NODES))
    h = jax.nn.relu(_gcn_conv(h, src, dst, w_e, W3, b3, N_NODES))
    h, src, dst, w_e, perm1, n1 = _topk_pool(h, src, dst, w_e, p1, N_NODES)
    h = jax.nn.relu(_gcn_conv(h, src, dst, w_e, W4, b4, n1))
    h = jax.nn.relu(_gcn_conv(h, src, dst, w_e, W5, b5, n1))
    h = jax.nn.relu(_gcn_conv(h, src, dst, w_e, W6, b6, n1))
    h, src, dst, w_e, perm2, n2 = _topk_pool(h, src, dst, w_e, p2, n1)

    logits = head(h, Wlin, blin)
    return (logits, perm1, perm2)
